# baseline (speedup 1.0000x reference)
import functools

import jax
import jax.numpy as jnp
from jax import lax
from jax.experimental import pallas as pl
from jax.experimental.pallas import tpu as pltpu
from jax.experimental.pallas import tpu_sc as plsc

N = 10000
E = 320000
D = 128
H = 128
G = 64
C = 40

NC = 2
NS = 16
NW = NC * NS
EPT = E // NW
K = 80
NCHUNK = EPT // K
KS = 128
CPT = 80
EPAD = NW * CPT * KS
NA = N + 8
ZR = 208
RPT = 624
TAIL_OFF = RPT * NS
TAIL = N - TAIL_OFF
BLK = 1000
NBLK = N // BLK

_mesh = functools.partial(
    plsc.VectorSubcoreMesh, core_axis_name="c", subcore_axis_name="s")


def _sc_degree(dst, zeros1, ones1):
    @functools.partial(
        pl.kernel,
        mesh=_mesh(),
        out_type=jax.ShapeDtypeStruct((NC * N,), jnp.float32),
        scratch_types=[
            pltpu.VMEM((K,), jnp.int32),
            pltpu.VMEM((K,), jnp.float32),
            pltpu.VMEM((RPT,), jnp.float32),
            pltpu.VMEM_SHARED((N,), jnp.float32),
        ],
    )
    def k(dst_hbm, z_hbm, o_hbm, out_hbm, didx, onesv, stage, acc):
        c = lax.axis_index("c")
        s = lax.axis_index("s")
        pltpu.sync_copy(z_hbm.at[pl.ds(0, RPT)], stage)
        pltpu.sync_copy(stage, acc.at[pl.ds(s * RPT, RPT)])

        @pl.when(s == 0)
        def _():
            pltpu.sync_copy(stage.at[pl.ds(0, TAIL)],
                            acc.at[pl.ds(TAIL_OFF, TAIL)])

        pltpu.sync_copy(o_hbm.at[pl.ds(0, K)], onesv)
        plsc.subcore_barrier()
        base = (c * NS + s) * EPT

        def body(i, carry):
            off = base + i * K
            pltpu.sync_copy(dst_hbm.at[pl.ds(off, K)], didx)
            pltpu.sync_copy(onesv, acc.at[didx], add=True)
            return carry

        lax.fori_loop(0, NCHUNK, body, 0)
        plsc.subcore_barrier()
        pltpu.sync_copy(acc.at[pl.ds(s * RPT, RPT)], stage)
        pltpu.sync_copy(stage, out_hbm.at[pl.ds(c * N + s * RPT, RPT)])

        @pl.when(s == 0)
        def _():
            pltpu.sync_copy(acc.at[pl.ds(TAIL_OFF, TAIL)],
                            onesv.at[pl.ds(0, TAIL)])
            pltpu.sync_copy(onesv.at[pl.ds(0, TAIL)],
                            out_hbm.at[pl.ds(c * N + TAIL_OFF, TAIL)])

    return k(dst, zeros1, ones1)


def _sc_scatter(hs, srcp, dstp, zeros_rows):
    @functools.partial(
        pl.kernel,
        mesh=_mesh(),
        out_type=jax.ShapeDtypeStruct((NC, N, D), jnp.float32),
        scratch_types=[
            pltpu.VMEM((KS,), jnp.int32),
            pltpu.VMEM((KS,), jnp.int32),
            pltpu.VMEM((KS,), jnp.int32),
            pltpu.VMEM((KS,), jnp.int32),
            pltpu.VMEM((KS, D), jnp.float32),
            pltpu.VMEM((KS, D), jnp.float32),
            pltpu.VMEM_SHARED((NA, D), jnp.float32),
            pltpu.SemaphoreType.DMA,
            pltpu.SemaphoreType.DMA,
        ],
    )
    def k(hs_hbm, src_hbm, dst_hbm, z_hbm, out_hbm, sidx0, sidx1,
          didx0, didx1, rows0, rows1, acc, gsem0, gsem1):
        c = lax.axis_index("c")
        s = lax.axis_index("s")
        tbase = (c * NS + s) * CPT * KS
        for q in range(RPT // ZR):
            pltpu.sync_copy(z_hbm, acc.at[pl.ds(s * RPT + q * ZR, ZR)])

        @pl.when(s == 0)
        def _():
            pltpu.sync_copy(z_hbm.at[pl.ds(0, NA - TAIL_OFF)],
                            acc.at[pl.ds(TAIL_OFF, NA - TAIL_OFF)])

        plsc.subcore_barrier()

        def body(j, carry):
            off = tbase + j * KS
            pltpu.sync_copy(src_hbm.at[pl.ds(off, KS)], sidx0)
            pltpu.sync_copy(dst_hbm.at[pl.ds(off, KS)], didx0)
            pltpu.async_copy(hs_hbm.at[sidx0], rows0, gsem0).wait()
            pltpu.sync_copy(rows0, acc.at[didx0], add=True)
            return carry

        lax.fori_loop(0, CPT, body, 0)
        del sidx1, didx1, rows1, gsem1
        plsc.subcore_barrier()
        pltpu.sync_copy(acc.at[pl.ds(s * RPT, RPT)],
                        out_hbm.at[c, pl.ds(s * RPT, RPT)])

        @pl.when(s == 0)
        def _():
            pltpu.sync_copy(acc.at[pl.ds(TAIL_OFF, TAIL)],
                            out_hbm.at[c, pl.ds(TAIL_OFF, TAIL)])

    return k(hs, srcp, dstp, zeros_rows)


def _tc_pre(deg2, x, W1):
    def body(deg_ref, x_ref, w_ref, dis_ref, hs_ref):
        deg = deg_ref[0] + deg_ref[1] + 1.0
        dis = lax.rsqrt(deg)
        dis_ref[...] = dis
        hw = jnp.dot(x_ref[...], w_ref[...],
                     preferred_element_type=jnp.float32)
        hs_ref[...] = dis * hw

    return pl.pallas_call(
        body,
        grid=(NBLK,),
        in_specs=[
            pl.BlockSpec((NC, BLK, 1), lambda i: (0, i, 0)),
            pl.BlockSpec((BLK, D), lambda i: (i, 0)),
            pl.BlockSpec((D, H), lambda i: (0, 0)),
        ],
        out_specs=[
            pl.BlockSpec((BLK, 1), lambda i: (i, 0)),
            pl.BlockSpec((BLK, H), lambda i: (i, 0)),
        ],
        out_shape=[
            jax.ShapeDtypeStruct((N, 1), jnp.float32),
            jax.ShapeDtypeStruct((N, H), jnp.float32),
        ],
    )(deg2, x, W1)


def _tc_mid(S2, hs, dis, prev, b, g, be, rm, rv, Wn, has_prev):
    def body(*refs):
        if has_prev:
            (s2_ref, hs_ref, dis_ref, prev_ref, b_ref, g_ref, be_ref,
             rm_ref, rv_ref, w_ref, h_ref, hsn_ref) = refs
        else:
            (s2_ref, hs_ref, dis_ref, b_ref, g_ref, be_ref,
             rm_ref, rv_ref, w_ref, h_ref, hsn_ref) = refs
        dis = dis_ref[...]
        z = dis * (s2_ref[0] + s2_ref[1] + hs_ref[...]) + b_ref[...]
        a = g_ref[...] * lax.rsqrt(rv_ref[...] + 1e-5)
        cst = be_ref[...] - rm_ref[...] * a
        h = z * a + cst
        if has_prev:
            h = h + prev_ref[...]
        h = jnp.maximum(h, 0.0)
        h_ref[...] = h
        hsn_ref[...] = dis * jnp.dot(h, w_ref[...],
                                     preferred_element_type=jnp.float32)

    in_specs = [
        pl.BlockSpec((NC, BLK, H), lambda i: (0, i, 0)),
        pl.BlockSpec((BLK, H), lambda i: (i, 0)),
        pl.BlockSpec((BLK, 1), lambda i: (i, 0)),
    ]
    args = [S2, hs, dis]
    if has_prev:
        in_specs.append(pl.BlockSpec((BLK, H), lambda i: (i, 0)))
        args.append(prev)
    in_specs += [pl.BlockSpec((1, H), lambda i: (0, 0))] * 5
    args += [b, g, be, rm, rv]
    in_specs.append(pl.BlockSpec((H, H), lambda i: (0, 0)))
    args.append(Wn)

    return pl.pallas_call(
        body,
        grid=(NBLK,),
        in_specs=in_specs,
        out_specs=[
            pl.BlockSpec((BLK, H), lambda i: (i, 0)),
            pl.BlockSpec((BLK, H), lambda i: (i, 0)),
        ],
        out_shape=[
            jax.ShapeDtypeStruct((N, H), jnp.float32),
            jax.ShapeDtypeStruct((N, H), jnp.float32),
        ],
    )(*args)


def _tc_final(S2, hs3, dis, h2, b3, g3, be3, rm3, rv3, batch2, Wo, bo):
    def body(s2_ref, hs_ref, dis_ref, prev_ref, b_ref, g_ref, be_ref,
             rm_ref, rv_ref, bat_ref, wo_ref, bo_ref, out_ref,
             s_acc, cnt_acc, mx_acc):
        i = pl.program_id(0)

        @pl.when(i == 0)
        def _():
            s_acc[...] = jnp.zeros((G, H), jnp.float32)
            cnt_acc[...] = jnp.zeros((G, H), jnp.float32)
            mx_acc[...] = jnp.full((G, H), -jnp.inf, jnp.float32)

        dis = dis_ref[...]
        z = dis * (s2_ref[0] + s2_ref[1] + hs_ref[...]) + b_ref[...]
        a = g_ref[...] * lax.rsqrt(rv_ref[...] + 1e-5)
        cst = be_ref[...] - rm_ref[...] * a
        h = jnp.maximum(z * a + cst + prev_ref[...], 0.0)

        bat = bat_ref[...]
        gids = lax.broadcasted_iota(jnp.int32, (BLK, G), 1)
        oh = (bat == gids).astype(jnp.float32)
        dn = (((0,), (0,)), ((), ()))
        s_acc[...] = s_acc[...] + lax.dot_general(
            oh, h, dn, preferred_element_type=jnp.float32)
        cnt_acc[...] = cnt_acc[...] + lax.dot_general(
            oh, jnp.ones((BLK, H), jnp.float32), dn,
            preferred_element_type=jnp.float32)

        g_lo = jnp.min(bat)
        g_hi = jnp.max(bat)

        def mbody(gg, carry):
            m = jnp.max(jnp.where(bat == gg, h, -jnp.inf), axis=0,
                        keepdims=True)
            mx_acc[pl.ds(gg, 1), :] = jnp.maximum(mx_acc[pl.ds(gg, 1), :], m)
            return carry

        lax.fori_loop(g_lo, g_hi + 1, mbody, 0)

        @pl.when(i == NBLK - 1)
        def _():
            cnt = jnp.maximum(cnt_acc[...], 1.0)
            mean = s_acc[...] / cnt
            pooled = jnp.concatenate([mean, s_acc[...], mx_acc[...]], axis=1)
            out_ref[...] = jnp.dot(pooled, wo_ref[...],
                                   preferred_element_type=jnp.float32
                                   ) + bo_ref[...]

    return pl.pallas_call(
        body,
        grid=(NBLK,),
        in_specs=[
            pl.BlockSpec((NC, BLK, H), lambda i: (0, i, 0)),
            pl.BlockSpec((BLK, H), lambda i: (i, 0)),
            pl.BlockSpec((BLK, 1), lambda i: (i, 0)),
            pl.BlockSpec((BLK, H), lambda i: (i, 0)),
        ] + [pl.BlockSpec((1, H), lambda i: (0, 0))] * 5 + [
            pl.BlockSpec((BLK, 1), lambda i: (i, 0)),
            pl.BlockSpec((3 * H, C), lambda i: (0, 0)),
            pl.BlockSpec((1, C), lambda i: (0, 0)),
        ],
        out_specs=pl.BlockSpec((G, C), lambda i: (0, 0)),
        out_shape=jax.ShapeDtypeStruct((G, C), jnp.float32),
        scratch_shapes=[
            pltpu.VMEM((G, H), jnp.float32),
            pltpu.VMEM((G, H), jnp.float32),
            pltpu.VMEM((G, H), jnp.float32),
        ],
    )(S2, hs3, dis, h2, b3, g3, be3, rm3, rv3, batch2, Wo, bo)


def kernel(x, edge_index, edge_attr, batch, W1, b1, W2, b2, W3, b3,
           g1, be1, rm1, rv1, g2, be2, rm2, rv2, g3, be3, rm3, rv3, Wo, bo):
    del edge_attr
    zeros1 = jnp.zeros((RPT + TAIL,), jnp.float32)
    ones1 = jnp.ones((K + 8,), jnp.float32)
    zrows = jnp.zeros((ZR, D), jnp.float32)
    r2 = lambda v: v.reshape(1, -1)
    batch2 = batch.reshape(N, 1)
    src = edge_index[0]
    dst = edge_index[1]
    srcp = jnp.concatenate([src, jnp.zeros((EPAD - E,), jnp.int32)])
    dstp = jnp.concatenate([dst, jnp.full((EPAD - E,), N, jnp.int32)])

    deg2 = _sc_degree(dst, zeros1, ones1).reshape(NC, N, 1)
    dis, hs1 = _tc_pre(deg2, x, W1)
    S1 = _sc_scatter(hs1, srcp, dstp, zrows)
    h1, hs2 = _tc_mid(S1, hs1, dis, None, r2(b1), r2(g1), r2(be1),
                      r2(rm1), r2(rv1), W2, has_prev=False)
    S2 = _sc_scatter(hs2, srcp, dstp, zrows)
    h2, hs3 = _tc_mid(S2, hs2, dis, h1, r2(b2), r2(g2), r2(be2),
                      r2(rm2), r2(rv2), W3, has_prev=True)
    S3 = _sc_scatter(hs3, srcp, dstp, zrows)
    out = _tc_final(S3, hs3, dis, h2, r2(b3), r2(g3), r2(be3),
                    r2(rm3), r2(rv3), batch2, Wo, r2(bo))
    return out


# speedup vs baseline: 1.2861x; 1.2861x over previous
import functools

import jax
import jax.numpy as jnp
from jax import lax
from jax.experimental import pallas as pl
from jax.experimental.pallas import tpu as pltpu
from jax.experimental.pallas import tpu_sc as plsc

N = 10000
E = 320000
D = 128
H = 128
G = 64
C = 40

NC = 2
NS = 16
NW = NC * NS
EPT = E // NW
K = 80
NCHUNK = EPT // K
KS = 128
CPT = 80
EPAD = NW * CPT * KS
NA = N + 8
ZR = 208
RPT = 624
TAIL_OFF = RPT * NS
TAIL = N - TAIL_OFF
BLK = 1000
NBLK = N // BLK

_mesh = functools.partial(
    plsc.VectorSubcoreMesh, core_axis_name="c", subcore_axis_name="s")


def _sc_degree(dst, zeros1, ones1):
    @functools.partial(
        pl.kernel,
        mesh=_mesh(),
        out_type=jax.ShapeDtypeStruct((NC * N,), jnp.float32),
        scratch_types=[
            pltpu.VMEM((K,), jnp.int32),
            pltpu.VMEM((K,), jnp.float32),
            pltpu.VMEM((RPT,), jnp.float32),
            pltpu.VMEM_SHARED((N,), jnp.float32),
        ],
    )
    def k(dst_hbm, z_hbm, o_hbm, out_hbm, didx, onesv, stage, acc):
        c = lax.axis_index("c")
        s = lax.axis_index("s")
        pltpu.sync_copy(z_hbm.at[pl.ds(0, RPT)], stage)
        pltpu.sync_copy(stage, acc.at[pl.ds(s * RPT, RPT)])

        @pl.when(s == 0)
        def _():
            pltpu.sync_copy(stage.at[pl.ds(0, TAIL)],
                            acc.at[pl.ds(TAIL_OFF, TAIL)])

        pltpu.sync_copy(o_hbm.at[pl.ds(0, K)], onesv)
        plsc.subcore_barrier()
        base = (c * NS + s) * EPT

        def body(i, carry):
            off = base + i * K
            pltpu.sync_copy(dst_hbm.at[pl.ds(off, K)], didx)
            pltpu.sync_copy(onesv, acc.at[didx], add=True)
            return carry

        lax.fori_loop(0, NCHUNK, body, 0)
        plsc.subcore_barrier()
        pltpu.sync_copy(acc.at[pl.ds(s * RPT, RPT)], stage)
        pltpu.sync_copy(stage, out_hbm.at[pl.ds(c * N + s * RPT, RPT)])

        @pl.when(s == 0)
        def _():
            pltpu.sync_copy(acc.at[pl.ds(TAIL_OFF, TAIL)],
                            onesv.at[pl.ds(0, TAIL)])
            pltpu.sync_copy(onesv.at[pl.ds(0, TAIL)],
                            out_hbm.at[pl.ds(c * N + TAIL_OFF, TAIL)])

    return k(dst, zeros1, ones1)


def _sc_scatter(hs, srcp, dstp, zeros_rows):
    @functools.partial(
        pl.kernel,
        mesh=_mesh(),
        out_type=jax.ShapeDtypeStruct((NC, N, D), jnp.float32),
        scratch_types=[
            pltpu.VMEM((KS,), jnp.int32),
            pltpu.VMEM((KS,), jnp.int32),
            pltpu.VMEM((KS,), jnp.int32),
            pltpu.VMEM((KS,), jnp.int32),
            pltpu.VMEM((KS, D), jnp.float32),
            pltpu.VMEM((KS, D), jnp.float32),
            pltpu.VMEM_SHARED((NA, D), jnp.float32),
            pltpu.SemaphoreType.DMA,
            pltpu.SemaphoreType.DMA,
        ],
    )
    def k(hs_hbm, src_hbm, dst_hbm, z_hbm, out_hbm, sidx0, sidx1,
          didx0, didx1, rows0, rows1, acc, gsem0, gsem1):
        c = lax.axis_index("c")
        s = lax.axis_index("s")
        tbase = (c * NS + s) * CPT * KS
        for q in range(RPT // ZR):
            pltpu.sync_copy(z_hbm, acc.at[pl.ds(s * RPT + q * ZR, ZR)])

        @pl.when(s == 0)
        def _():
            pltpu.sync_copy(z_hbm.at[pl.ds(0, NA - TAIL_OFF)],
                            acc.at[pl.ds(TAIL_OFF, NA - TAIL_OFF)])

        plsc.subcore_barrier()

        bufs = (rows0, rows1)
        sems = (gsem0, gsem1)
        sidxs = (sidx0, sidx1)
        didxs = (didx0, didx1)

        def load_idx(j, b):
            off = tbase + j * KS
            pltpu.sync_copy(src_hbm.at[pl.ds(off, KS)], sidxs[b])
            pltpu.sync_copy(dst_hbm.at[pl.ds(off, KS)], didxs[b])

        for b in range(2):
            load_idx(b, b)
            pltpu.async_copy(hs_hbm.at[sidxs[b]], bufs[b], sems[b])

        def body(p, carry):
            for b in range(2):
                j = 2 * p + b
                pltpu.make_async_copy(
                    hs_hbm.at[sidxs[b]], bufs[b], sems[b]).wait()
                pltpu.sync_copy(bufs[b], acc.at[didxs[b]], add=True)

                @pl.when(j + 2 < CPT)
                def _():
                    load_idx(j + 2, b)
                    pltpu.async_copy(hs_hbm.at[sidxs[b]], bufs[b], sems[b])
            return carry

        lax.fori_loop(0, CPT // 2, body, 0)
        plsc.subcore_barrier()
        pltpu.sync_copy(acc.at[pl.ds(s * RPT, RPT)],
                        out_hbm.at[c, pl.ds(s * RPT, RPT)])

        @pl.when(s == 0)
        def _():
            pltpu.sync_copy(acc.at[pl.ds(TAIL_OFF, TAIL)],
                            out_hbm.at[c, pl.ds(TAIL_OFF, TAIL)])

    return k(hs, srcp, dstp, zeros_rows)


def _tc_pre(deg2, x, W1):
    def body(deg_ref, x_ref, w_ref, dis_ref, hs_ref):
        deg = deg_ref[0] + deg_ref[1] + 1.0
        dis = lax.rsqrt(deg)
        dis_ref[...] = dis
        hw = jnp.dot(x_ref[...], w_ref[...],
                     preferred_element_type=jnp.float32)
        hs_ref[...] = dis * hw

    return pl.pallas_call(
        body,
        grid=(NBLK,),
        in_specs=[
            pl.BlockSpec((NC, BLK, 1), lambda i: (0, i, 0)),
            pl.BlockSpec((BLK, D), lambda i: (i, 0)),
            pl.BlockSpec((D, H), lambda i: (0, 0)),
        ],
        out_specs=[
            pl.BlockSpec((BLK, 1), lambda i: (i, 0)),
            pl.BlockSpec((BLK, H), lambda i: (i, 0)),
        ],
        out_shape=[
            jax.ShapeDtypeStruct((N, 1), jnp.float32),
            jax.ShapeDtypeStruct((N, H), jnp.float32),
        ],
    )(deg2, x, W1)


def _tc_mid(S2, hs, dis, prev, b, g, be, rm, rv, Wn, has_prev):
    def body(*refs):
        if has_prev:
            (s2_ref, hs_ref, dis_ref, prev_ref, b_ref, g_ref, be_ref,
             rm_ref, rv_ref, w_ref, h_ref, hsn_ref) = refs
        else:
            (s2_ref, hs_ref, dis_ref, b_ref, g_ref, be_ref,
             rm_ref, rv_ref, w_ref, h_ref, hsn_ref) = refs
        dis = dis_ref[...]
        z = dis * (s2_ref[0] + s2_ref[1] + hs_ref[...]) + b_ref[...]
        a = g_ref[...] * lax.rsqrt(rv_ref[...] + 1e-5)
        cst = be_ref[...] - rm_ref[...] * a
        h = z * a + cst
        if has_prev:
            h = h + prev_ref[...]
        h = jnp.maximum(h, 0.0)
        h_ref[...] = h
        hsn_ref[...] = dis * jnp.dot(h, w_ref[...],
                                     preferred_element_type=jnp.float32)

    in_specs = [
        pl.BlockSpec((NC, BLK, H), lambda i: (0, i, 0)),
        pl.BlockSpec((BLK, H), lambda i: (i, 0)),
        pl.BlockSpec((BLK, 1), lambda i: (i, 0)),
    ]
    args = [S2, hs, dis]
    if has_prev:
        in_specs.append(pl.BlockSpec((BLK, H), lambda i: (i, 0)))
        args.append(prev)
    in_specs += [pl.BlockSpec((1, H), lambda i: (0, 0))] * 5
    args += [b, g, be, rm, rv]
    in_specs.append(pl.BlockSpec((H, H), lambda i: (0, 0)))
    args.append(Wn)

    return pl.pallas_call(
        body,
        grid=(NBLK,),
        in_specs=in_specs,
        out_specs=[
            pl.BlockSpec((BLK, H), lambda i: (i, 0)),
            pl.BlockSpec((BLK, H), lambda i: (i, 0)),
        ],
        out_shape=[
            jax.ShapeDtypeStruct((N, H), jnp.float32),
            jax.ShapeDtypeStruct((N, H), jnp.float32),
        ],
    )(*args)


def _tc_final(S2, hs3, dis, h2, b3, g3, be3, rm3, rv3, batch2, Wo, bo):
    def body(s2_ref, hs_ref, dis_ref, prev_ref, b_ref, g_ref, be_ref,
             rm_ref, rv_ref, bat_ref, wo_ref, bo_ref, out_ref,
             s_acc, cnt_acc, mx_acc):
        i = pl.program_id(0)

        @pl.when(i == 0)
        def _():
            s_acc[...] = jnp.zeros((G, H), jnp.float32)
            cnt_acc[...] = jnp.zeros((G, H), jnp.float32)
            mx_acc[...] = jnp.full((G, H), -jnp.inf, jnp.float32)

        dis = dis_ref[...]
        z = dis * (s2_ref[0] + s2_ref[1] + hs_ref[...]) + b_ref[...]
        a = g_ref[...] * lax.rsqrt(rv_ref[...] + 1e-5)
        cst = be_ref[...] - rm_ref[...] * a
        h = jnp.maximum(z * a + cst + prev_ref[...], 0.0)

        bat = bat_ref[...]
        gids = lax.broadcasted_iota(jnp.int32, (BLK, G), 1)
        oh = (bat == gids).astype(jnp.float32)
        dn = (((0,), (0,)), ((), ()))
        s_acc[...] = s_acc[...] + lax.dot_general(
            oh, h, dn, preferred_element_type=jnp.float32)
        cnt_acc[...] = cnt_acc[...] + lax.dot_general(
            oh, jnp.ones((BLK, H), jnp.float32), dn,
            preferred_element_type=jnp.float32)

        g_lo = jnp.min(bat)
        g_hi = jnp.max(bat)

        def mbody(gg, carry):
            m = jnp.max(jnp.where(bat == gg, h, -jnp.inf), axis=0,
                        keepdims=True)
            mx_acc[pl.ds(gg, 1), :] = jnp.maximum(mx_acc[pl.ds(gg, 1), :], m)
            return carry

        lax.fori_loop(g_lo, g_hi + 1, mbody, 0)

        @pl.when(i == NBLK - 1)
        def _():
            cnt = jnp.maximum(cnt_acc[...], 1.0)
            mean = s_acc[...] / cnt
            pooled = jnp.concatenate([mean, s_acc[...], mx_acc[...]], axis=1)
            out_ref[...] = jnp.dot(pooled, wo_ref[...],
                                   preferred_element_type=jnp.float32
                                   ) + bo_ref[...]

    return pl.pallas_call(
        body,
        grid=(NBLK,),
        in_specs=[
            pl.BlockSpec((NC, BLK, H), lambda i: (0, i, 0)),
            pl.BlockSpec((BLK, H), lambda i: (i, 0)),
            pl.BlockSpec((BLK, 1), lambda i: (i, 0)),
            pl.BlockSpec((BLK, H), lambda i: (i, 0)),
        ] + [pl.BlockSpec((1, H), lambda i: (0, 0))] * 5 + [
            pl.BlockSpec((BLK, 1), lambda i: (i, 0)),
            pl.BlockSpec((3 * H, C), lambda i: (0, 0)),
            pl.BlockSpec((1, C), lambda i: (0, 0)),
        ],
        out_specs=pl.BlockSpec((G, C), lambda i: (0, 0)),
        out_shape=jax.ShapeDtypeStruct((G, C), jnp.float32),
        scratch_shapes=[
            pltpu.VMEM((G, H), jnp.float32),
            pltpu.VMEM((G, H), jnp.float32),
            pltpu.VMEM((G, H), jnp.float32),
        ],
    )(S2, hs3, dis, h2, b3, g3, be3, rm3, rv3, batch2, Wo, bo)


def kernel(x, edge_index, edge_attr, batch, W1, b1, W2, b2, W3, b3,
           g1, be1, rm1, rv1, g2, be2, rm2, rv2, g3, be3, rm3, rv3, Wo, bo):
    del edge_attr
    zeros1 = jnp.zeros((RPT + TAIL,), jnp.float32)
    ones1 = jnp.ones((K + 8,), jnp.float32)
    zrows = jnp.zeros((ZR, D), jnp.float32)
    r2 = lambda v: v.reshape(1, -1)
    batch2 = batch.reshape(N, 1)
    src = edge_index[0]
    dst = edge_index[1]
    PPT = CPT * KS - EPT
    srcp = jnp.concatenate(
        [src.reshape(NW, EPT),
         jnp.zeros((NW, PPT), jnp.int32)], axis=1).reshape(-1)
    pad_dst = N + (jnp.arange(PPT, dtype=jnp.int32) % 8)
    dstp = jnp.concatenate(
        [dst.reshape(NW, EPT),
         jnp.broadcast_to(pad_dst, (NW, PPT))], axis=1).reshape(-1)

    deg2 = _sc_degree(dst, zeros1, ones1).reshape(NC, N, 1)
    dis, hs1 = _tc_pre(deg2, x, W1)
    S1 = _sc_scatter(hs1, srcp, dstp, zrows)
    h1, hs2 = _tc_mid(S1, hs1, dis, None, r2(b1), r2(g1), r2(be1),
                      r2(rm1), r2(rv1), W2, has_prev=False)
    S2 = _sc_scatter(hs2, srcp, dstp, zrows)
    h2, hs3 = _tc_mid(S2, hs2, dis, h1, r2(b2), r2(g2), r2(be2),
                      r2(rm2), r2(rv2), W3, has_prev=True)
    S3 = _sc_scatter(hs3, srcp, dstp, zrows)
    out = _tc_final(S3, hs3, dis, h2, r2(b3), r2(g3), r2(be3),
                    r2(rm3), r2(rv3), batch2, Wo, r2(bo))
    return out


# speedup vs baseline: 2.5838x; 2.0090x over previous
import functools

import jax
import jax.numpy as jnp
from jax import lax
from jax.experimental import pallas as pl
from jax.experimental.pallas import tpu as pltpu
from jax.experimental.pallas import tpu_sc as plsc

N = 10000
E = 320000
D = 128
H = 128
G = 64
C = 40

NC = 2
NS = 16
NW = NC * NS
EPT = E // NW
K = 80
NCHUNK = EPT // K
KS = 128
CPT = 80
EPAD = NW * CPT * KS
NA = N + 8
ZR = 208
RPT = 624
TAIL_OFF = RPT * NS
TAIL = N - TAIL_OFF
BLK = 1000
NBLK = N // BLK

_mesh = functools.partial(
    plsc.VectorSubcoreMesh, core_axis_name="c", subcore_axis_name="s")


def _sc_degree(dst, zeros1, ones1):
    @functools.partial(
        pl.kernel,
        mesh=_mesh(),
        out_type=jax.ShapeDtypeStruct((NC * N,), jnp.float32),
        scratch_types=[
            pltpu.VMEM((K,), jnp.int32),
            pltpu.VMEM((K,), jnp.float32),
            pltpu.VMEM((RPT,), jnp.float32),
            pltpu.VMEM_SHARED((N,), jnp.float32),
        ],
    )
    def k(dst_hbm, z_hbm, o_hbm, out_hbm, didx, onesv, stage, acc):
        c = lax.axis_index("c")
        s = lax.axis_index("s")
        pltpu.sync_copy(z_hbm.at[pl.ds(0, RPT)], stage)
        pltpu.sync_copy(stage, acc.at[pl.ds(s * RPT, RPT)])

        @pl.when(s == 0)
        def _():
            pltpu.sync_copy(stage.at[pl.ds(0, TAIL)],
                            acc.at[pl.ds(TAIL_OFF, TAIL)])

        pltpu.sync_copy(o_hbm.at[pl.ds(0, K)], onesv)
        plsc.subcore_barrier()
        base = (c * NS + s) * EPT

        def body(i, carry):
            off = base + i * K
            pltpu.sync_copy(dst_hbm.at[pl.ds(off, K)], didx)
            pltpu.sync_copy(onesv, acc.at[didx], add=True)
            return carry

        lax.fori_loop(0, NCHUNK, body, 0)
        plsc.subcore_barrier()
        pltpu.sync_copy(acc.at[pl.ds(s * RPT, RPT)], stage)
        pltpu.sync_copy(stage, out_hbm.at[pl.ds(c * N + s * RPT, RPT)])

        @pl.when(s == 0)
        def _():
            pltpu.sync_copy(acc.at[pl.ds(TAIL_OFF, TAIL)],
                            onesv.at[pl.ds(0, TAIL)])
            pltpu.sync_copy(onesv.at[pl.ds(0, TAIL)],
                            out_hbm.at[pl.ds(c * N + TAIL_OFF, TAIL)])

    return k(dst, zeros1, ones1)


def _sc_scatter(hs, srcp, dstp, zeros_rows):
    @functools.partial(
        pl.kernel,
        mesh=_mesh(),
        out_type=jax.ShapeDtypeStruct((NC, N, D), jnp.float32),
        scratch_types=[
            pltpu.VMEM((K,), jnp.int32),
            pltpu.VMEM((K,), jnp.int32),
            pltpu.VMEM((K,), jnp.int32),
            pltpu.VMEM((K,), jnp.int32),
            pltpu.VMEM((K, D), jnp.float32),
            pltpu.VMEM((K, D), jnp.float32),
            pltpu.VMEM_SHARED((NA, D), jnp.float32),
            pltpu.SemaphoreType.DMA,
            pltpu.SemaphoreType.DMA,
        ],
    )
    def k(hs_hbm, src_hbm, dst_hbm, z_hbm, out_hbm, sidx0, sidx1,
          didx0, didx1, rows0, rows1, acc, gsem0, gsem1):
        c = lax.axis_index("c")
        s = lax.axis_index("s")
        tbase = (c * NS + s) * EPT
        for q in range(RPT // ZR):
            pltpu.sync_copy(z_hbm, acc.at[pl.ds(s * RPT + q * ZR, ZR)])

        @pl.when(s == 0)
        def _():
            pltpu.sync_copy(z_hbm.at[pl.ds(0, NA - TAIL_OFF)],
                            acc.at[pl.ds(TAIL_OFF, NA - TAIL_OFF)])

        plsc.subcore_barrier()

        bufs = (rows0, rows1)
        sems = (gsem0, gsem1)
        sidxs = (sidx0, sidx1)
        didxs = (didx0, didx1)

        def load_idx(j, b):
            off = tbase + j * K
            pltpu.sync_copy(src_hbm.at[pl.ds(off, K)], sidxs[b])
            pltpu.sync_copy(dst_hbm.at[pl.ds(off, K)], didxs[b])

        for b in range(2):
            load_idx(b, b)
            pltpu.async_copy(hs_hbm.at[sidxs[b]], bufs[b], sems[b])

        def body(p, carry):
            for b in range(2):
                j = 2 * p + b
                pltpu.make_async_copy(
                    hs_hbm.at[sidxs[b]], bufs[b], sems[b]).wait()
                pltpu.sync_copy(bufs[b], acc.at[didxs[b]], add=True)

                @pl.when(j + 2 < NCHUNK)
                def _():
                    load_idx(j + 2, b)
                    pltpu.async_copy(hs_hbm.at[sidxs[b]], bufs[b], sems[b])
            return carry

        lax.fori_loop(0, NCHUNK // 2, body, 0)
        pltpu.make_async_copy(hs_hbm.at[sidx0], rows0, gsem0).wait()
        pltpu.sync_copy(rows0, acc.at[didx0], add=True)
        plsc.subcore_barrier()
        pltpu.sync_copy(acc.at[pl.ds(s * RPT, RPT)],
                        out_hbm.at[c, pl.ds(s * RPT, RPT)])

        @pl.when(s == 0)
        def _():
            pltpu.sync_copy(acc.at[pl.ds(TAIL_OFF, TAIL)],
                            out_hbm.at[c, pl.ds(TAIL_OFF, TAIL)])

    return k(hs, srcp, dstp, zeros_rows)


def _tc_pre(deg2, x, W1):
    def body(deg_ref, x_ref, w_ref, dis_ref, hs_ref):
        deg = deg_ref[0] + deg_ref[1] + 1.0
        dis = lax.rsqrt(deg)
        dis_ref[...] = dis
        hw = jnp.dot(x_ref[...], w_ref[...],
                     preferred_element_type=jnp.float32)
        hs_ref[...] = dis * hw

    return pl.pallas_call(
        body,
        grid=(NBLK,),
        in_specs=[
            pl.BlockSpec((NC, BLK, 1), lambda i: (0, i, 0)),
            pl.BlockSpec((BLK, D), lambda i: (i, 0)),
            pl.BlockSpec((D, H), lambda i: (0, 0)),
        ],
        out_specs=[
            pl.BlockSpec((BLK, 1), lambda i: (i, 0)),
            pl.BlockSpec((BLK, H), lambda i: (i, 0)),
        ],
        out_shape=[
            jax.ShapeDtypeStruct((N, 1), jnp.float32),
            jax.ShapeDtypeStruct((N, H), jnp.float32),
        ],
    )(deg2, x, W1)


def _tc_mid(S2, hs, dis, prev, b, g, be, rm, rv, Wn, has_prev):
    def body(*refs):
        if has_prev:
            (s2_ref, hs_ref, dis_ref, prev_ref, b_ref, g_ref, be_ref,
             rm_ref, rv_ref, w_ref, h_ref, hsn_ref) = refs
        else:
            (s2_ref, hs_ref, dis_ref, b_ref, g_ref, be_ref,
             rm_ref, rv_ref, w_ref, h_ref, hsn_ref) = refs
        dis = dis_ref[...]
        z = dis * (s2_ref[0] + s2_ref[1] + hs_ref[...]) + b_ref[...]
        a = g_ref[...] * lax.rsqrt(rv_ref[...] + 1e-5)
        cst = be_ref[...] - rm_ref[...] * a
        h = z * a + cst
        if has_prev:
            h = h + prev_ref[...]
        h = jnp.maximum(h, 0.0)
        h_ref[...] = h
        hsn_ref[...] = dis * jnp.dot(h, w_ref[...],
                                     preferred_element_type=jnp.float32)

    in_specs = [
        pl.BlockSpec((NC, BLK, H), lambda i: (0, i, 0)),
        pl.BlockSpec((BLK, H), lambda i: (i, 0)),
        pl.BlockSpec((BLK, 1), lambda i: (i, 0)),
    ]
    args = [S2, hs, dis]
    if has_prev:
        in_specs.append(pl.BlockSpec((BLK, H), lambda i: (i, 0)))
        args.append(prev)
    in_specs += [pl.BlockSpec((1, H), lambda i: (0, 0))] * 5
    args += [b, g, be, rm, rv]
    in_specs.append(pl.BlockSpec((H, H), lambda i: (0, 0)))
    args.append(Wn)

    return pl.pallas_call(
        body,
        grid=(NBLK,),
        in_specs=in_specs,
        out_specs=[
            pl.BlockSpec((BLK, H), lambda i: (i, 0)),
            pl.BlockSpec((BLK, H), lambda i: (i, 0)),
        ],
        out_shape=[
            jax.ShapeDtypeStruct((N, H), jnp.float32),
            jax.ShapeDtypeStruct((N, H), jnp.float32),
        ],
    )(*args)


def _tc_final(S2, hs3, dis, h2, b3, g3, be3, rm3, rv3, batch2, Wo, bo):
    def body(s2_ref, hs_ref, dis_ref, prev_ref, b_ref, g_ref, be_ref,
             rm_ref, rv_ref, bat_ref, wo_ref, bo_ref, out_ref,
             s_acc, cnt_acc, mx_acc):
        i = pl.program_id(0)

        @pl.when(i == 0)
        def _():
            s_acc[...] = jnp.zeros((G, H), jnp.float32)
            cnt_acc[...] = jnp.zeros((G, H), jnp.float32)
            mx_acc[...] = jnp.full((G, H), -jnp.inf, jnp.float32)

        dis = dis_ref[...]
        z = dis * (s2_ref[0] + s2_ref[1] + hs_ref[...]) + b_ref[...]
        a = g_ref[...] * lax.rsqrt(rv_ref[...] + 1e-5)
        cst = be_ref[...] - rm_ref[...] * a
        h = jnp.maximum(z * a + cst + prev_ref[...], 0.0)

        bat = bat_ref[...]
        gids = lax.broadcasted_iota(jnp.int32, (BLK, G), 1)
        oh = (bat == gids).astype(jnp.float32)
        dn = (((0,), (0,)), ((), ()))
        s_acc[...] = s_acc[...] + lax.dot_general(
            oh, h, dn, preferred_element_type=jnp.float32)
        cnt_acc[...] = cnt_acc[...] + lax.dot_general(
            oh, jnp.ones((BLK, H), jnp.float32), dn,
            preferred_element_type=jnp.float32)

        g_lo = jnp.min(bat)
        g_hi = jnp.max(bat)

        def mbody(gg, carry):
            m = jnp.max(jnp.where(bat == gg, h, -jnp.inf), axis=0,
                        keepdims=True)
            mx_acc[pl.ds(gg, 1), :] = jnp.maximum(mx_acc[pl.ds(gg, 1), :], m)
            return carry

        lax.fori_loop(g_lo, g_hi + 1, mbody, 0)

        @pl.when(i == NBLK - 1)
        def _():
            cnt = jnp.maximum(cnt_acc[...], 1.0)
            mean = s_acc[...] / cnt
            pooled = jnp.concatenate([mean, s_acc[...], mx_acc[...]], axis=1)
            out_ref[...] = jnp.dot(pooled, wo_ref[...],
                                   preferred_element_type=jnp.float32
                                   ) + bo_ref[...]

    return pl.pallas_call(
        body,
        grid=(NBLK,),
        in_specs=[
            pl.BlockSpec((NC, BLK, H), lambda i: (0, i, 0)),
            pl.BlockSpec((BLK, H), lambda i: (i, 0)),
            pl.BlockSpec((BLK, 1), lambda i: (i, 0)),
            pl.BlockSpec((BLK, H), lambda i: (i, 0)),
        ] + [pl.BlockSpec((1, H), lambda i: (0, 0))] * 5 + [
            pl.BlockSpec((BLK, 1), lambda i: (i, 0)),
            pl.BlockSpec((3 * H, C), lambda i: (0, 0)),
            pl.BlockSpec((1, C), lambda i: (0, 0)),
        ],
        out_specs=pl.BlockSpec((G, C), lambda i: (0, 0)),
        out_shape=jax.ShapeDtypeStruct((G, C), jnp.float32),
        scratch_shapes=[
            pltpu.VMEM((G, H), jnp.float32),
            pltpu.VMEM((G, H), jnp.float32),
            pltpu.VMEM((G, H), jnp.float32),
        ],
    )(S2, hs3, dis, h2, b3, g3, be3, rm3, rv3, batch2, Wo, bo)


def kernel(x, edge_index, edge_attr, batch, W1, b1, W2, b2, W3, b3,
           g1, be1, rm1, rv1, g2, be2, rm2, rv2, g3, be3, rm3, rv3, Wo, bo):
    del edge_attr
    zeros1 = jnp.zeros((RPT + TAIL,), jnp.float32)
    ones1 = jnp.ones((K + 8,), jnp.float32)
    zrows = jnp.zeros((ZR, D), jnp.float32)
    r2 = lambda v: v.reshape(1, -1)
    batch2 = batch.reshape(N, 1)
    src = edge_index[0]
    dst = edge_index[1]
    srcp = src
    dstp = dst

    deg2 = _sc_degree(dst, zeros1, ones1).reshape(NC, N, 1)
    dis, hs1 = _tc_pre(deg2, x, W1)
    S1 = _sc_scatter(hs1, srcp, dstp, zrows)
    h1, hs2 = _tc_mid(S1, hs1, dis, None, r2(b1), r2(g1), r2(be1),
                      r2(rm1), r2(rv1), W2, has_prev=False)
    S2 = _sc_scatter(hs2, srcp, dstp, zrows)
    h2, hs3 = _tc_mid(S2, hs2, dis, h1, r2(b2), r2(g2), r2(be2),
                      r2(rm2), r2(rv2), W3, has_prev=True)
    S3 = _sc_scatter(hs3, srcp, dstp, zrows)
    out = _tc_final(S3, hs3, dis, h2, r2(b3), r2(g3), r2(be3),
                    r2(rm3), r2(rv3), batch2, Wo, r2(bo))
    return out


# speedup vs baseline: 3.2015x; 1.2391x over previous
import functools

import jax
import jax.numpy as jnp
from jax import lax
from jax.experimental import pallas as pl
from jax.experimental.pallas import tpu as pltpu
from jax.experimental.pallas import tpu_sc as plsc

N = 10000
E = 320000
D = 128
H = 128
G = 64
C = 40

NC = 2
NS = 16
NW = NC * NS
EPT = E // NW
K = 80
NCHUNK = EPT // K
KS = 128
CPT = 80
EPAD = NW * CPT * KS
NA = N + 8
ZR = 208
RPT = 624
TAIL_OFF = RPT * NS
TAIL = N - TAIL_OFF
BLK = 1000
NBLK = N // BLK

_mesh = functools.partial(
    plsc.VectorSubcoreMesh, core_axis_name="c", subcore_axis_name="s")


def _sc_degree(dst, zeros1, ones1):
    @functools.partial(
        pl.kernel,
        mesh=_mesh(),
        out_type=jax.ShapeDtypeStruct((NC * N,), jnp.float32),
        scratch_types=[
            pltpu.VMEM((K,), jnp.int32),
            pltpu.VMEM((K,), jnp.int32),
            pltpu.VMEM((K,), jnp.float32),
            pltpu.VMEM((RPT,), jnp.float32),
            pltpu.VMEM_SHARED((N,), jnp.float32),
            pltpu.SemaphoreType.DMA,
            pltpu.SemaphoreType.DMA,
        ],
    )
    def k(dst_hbm, z_hbm, o_hbm, out_hbm, didx0, didx1, onesv, stage, acc,
          isem0, isem1):
        c = lax.axis_index("c")
        s = lax.axis_index("s")
        pltpu.sync_copy(z_hbm.at[pl.ds(0, RPT)], stage)
        pltpu.sync_copy(stage, acc.at[pl.ds(s * RPT, RPT)])

        @pl.when(s == 0)
        def _():
            pltpu.sync_copy(stage.at[pl.ds(0, TAIL)],
                            acc.at[pl.ds(TAIL_OFF, TAIL)])

        pltpu.sync_copy(o_hbm.at[pl.ds(0, K)], onesv)
        plsc.subcore_barrier()
        base = (c * NS + s) * EPT
        didxs = (didx0, didx1)
        isems = (isem0, isem1)

        def idx_start(j, b):
            pltpu.async_copy(dst_hbm.at[pl.ds(base + j * K, K)],
                             didxs[b], isems[b])

        def idx_wait(j, b):
            pltpu.make_async_copy(dst_hbm.at[pl.ds(base + j * K, K)],
                                  didxs[b], isems[b]).wait()

        idx_start(0, 0)
        idx_start(1, 1)

        def body(p, carry):
            for b in range(2):
                j = 2 * p + b
                idx_wait(j, b)
                pltpu.sync_copy(onesv, acc.at[didxs[b]], add=True)

                @pl.when(j + 2 < NCHUNK)
                def _():
                    idx_start(j + 2, b)
            return carry

        lax.fori_loop(0, NCHUNK // 2, body, 0)
        idx_wait(NCHUNK - 1, 0)
        pltpu.sync_copy(onesv, acc.at[didx0], add=True)
        plsc.subcore_barrier()
        pltpu.sync_copy(acc.at[pl.ds(s * RPT, RPT)], stage)
        pltpu.sync_copy(stage, out_hbm.at[pl.ds(c * N + s * RPT, RPT)])

        @pl.when(s == 0)
        def _():
            pltpu.sync_copy(acc.at[pl.ds(TAIL_OFF, TAIL)],
                            onesv.at[pl.ds(0, TAIL)])
            pltpu.sync_copy(onesv.at[pl.ds(0, TAIL)],
                            out_hbm.at[pl.ds(c * N + TAIL_OFF, TAIL)])

    return k(dst, zeros1, ones1)


def _sc_scatter(hs, srcp, dstp, zeros_rows):
    @functools.partial(
        pl.kernel,
        mesh=_mesh(),
        out_type=jax.ShapeDtypeStruct((NC, N, D), jnp.float32),
        scratch_types=[
            pltpu.VMEM((K,), jnp.int32),
            pltpu.VMEM((K,), jnp.int32),
            pltpu.VMEM((K,), jnp.int32),
            pltpu.VMEM((K,), jnp.int32),
            pltpu.VMEM((K, D), jnp.float32),
            pltpu.VMEM((K, D), jnp.float32),
            pltpu.VMEM_SHARED((NA, D), jnp.float32),
            pltpu.SemaphoreType.DMA,
            pltpu.SemaphoreType.DMA,
            pltpu.SemaphoreType.DMA,
            pltpu.SemaphoreType.DMA,
        ],
    )
    def k(hs_hbm, src_hbm, dst_hbm, z_hbm, out_hbm, sidx0, sidx1,
          didx0, didx1, rows0, rows1, acc, gsem0, gsem1, isem0, isem1):
        c = lax.axis_index("c")
        s = lax.axis_index("s")
        tbase = (c * NS + s) * EPT
        for q in range(RPT // ZR):
            pltpu.sync_copy(z_hbm, acc.at[pl.ds(s * RPT + q * ZR, ZR)])

        @pl.when(s == 0)
        def _():
            pltpu.sync_copy(z_hbm.at[pl.ds(0, NA - TAIL_OFF)],
                            acc.at[pl.ds(TAIL_OFF, NA - TAIL_OFF)])

        plsc.subcore_barrier()

        bufs = (rows0, rows1)
        sems = (gsem0, gsem1)
        isems = (isem0, isem1)
        sidxs = (sidx0, sidx1)
        didxs = (didx0, didx1)

        def idx_start(j, b):
            off = tbase + j * K
            pltpu.async_copy(src_hbm.at[pl.ds(off, K)], sidxs[b], isems[b])
            pltpu.async_copy(dst_hbm.at[pl.ds(off, K)], didxs[b], isems[b])

        def idx_wait(j, b):
            off = tbase + j * K
            pltpu.make_async_copy(
                src_hbm.at[pl.ds(off, K)], sidxs[b], isems[b]).wait()
            pltpu.make_async_copy(
                dst_hbm.at[pl.ds(off, K)], didxs[b], isems[b]).wait()

        idx_start(0, 0)
        idx_start(1, 1)
        idx_wait(0, 0)
        pltpu.async_copy(hs_hbm.at[sidx0], rows0, gsem0)

        def body(p, carry):
            for b in range(2):
                j = 2 * p + b
                b1 = 1 - b

                @pl.when(j + 1 < NCHUNK)
                def _():
                    idx_wait(j + 1, b1)
                    pltpu.async_copy(hs_hbm.at[sidxs[b1]], bufs[b1],
                                     sems[b1])

                pltpu.make_async_copy(
                    hs_hbm.at[sidxs[b]], bufs[b], sems[b]).wait()
                pltpu.sync_copy(bufs[b], acc.at[didxs[b]], add=True)

                @pl.when(j + 2 < NCHUNK)
                def _():
                    idx_start(j + 2, b)
            return carry

        lax.fori_loop(0, NCHUNK // 2, body, 0)
        pltpu.make_async_copy(hs_hbm.at[sidx0], rows0, gsem0).wait()
        pltpu.sync_copy(rows0, acc.at[didx0], add=True)
        plsc.subcore_barrier()
        pltpu.sync_copy(acc.at[pl.ds(s * RPT, RPT)],
                        out_hbm.at[c, pl.ds(s * RPT, RPT)])

        @pl.when(s == 0)
        def _():
            pltpu.sync_copy(acc.at[pl.ds(TAIL_OFF, TAIL)],
                            out_hbm.at[c, pl.ds(TAIL_OFF, TAIL)])

    return k(hs, srcp, dstp, zeros_rows)


def _tc_pre(deg2, x, W1):
    def body(deg_ref, x_ref, w_ref, dis_ref, hs_ref):
        deg = deg_ref[0] + deg_ref[1] + 1.0
        dis = lax.rsqrt(deg)
        dis_ref[...] = dis
        hw = jnp.dot(x_ref[...], w_ref[...],
                     preferred_element_type=jnp.float32)
        hs_ref[...] = dis * hw

    return pl.pallas_call(
        body,
        grid=(NBLK,),
        in_specs=[
            pl.BlockSpec((NC, BLK, 1), lambda i: (0, i, 0)),
            pl.BlockSpec((BLK, D), lambda i: (i, 0)),
            pl.BlockSpec((D, H), lambda i: (0, 0)),
        ],
        out_specs=[
            pl.BlockSpec((BLK, 1), lambda i: (i, 0)),
            pl.BlockSpec((BLK, H), lambda i: (i, 0)),
        ],
        out_shape=[
            jax.ShapeDtypeStruct((N, 1), jnp.float32),
            jax.ShapeDtypeStruct((N, H), jnp.float32),
        ],
    )(deg2, x, W1)


def _tc_mid(S2, hs, dis, prev, b, g, be, rm, rv, Wn, has_prev):
    def body(*refs):
        if has_prev:
            (s2_ref, hs_ref, dis_ref, prev_ref, b_ref, g_ref, be_ref,
             rm_ref, rv_ref, w_ref, h_ref, hsn_ref) = refs
        else:
            (s2_ref, hs_ref, dis_ref, b_ref, g_ref, be_ref,
             rm_ref, rv_ref, w_ref, h_ref, hsn_ref) = refs
        dis = dis_ref[...]
        z = dis * (s2_ref[0] + s2_ref[1] + hs_ref[...]) + b_ref[...]
        a = g_ref[...] * lax.rsqrt(rv_ref[...] + 1e-5)
        cst = be_ref[...] - rm_ref[...] * a
        h = z * a + cst
        if has_prev:
            h = h + prev_ref[...]
        h = jnp.maximum(h, 0.0)
        h_ref[...] = h
        hsn_ref[...] = dis * jnp.dot(h, w_ref[...],
                                     preferred_element_type=jnp.float32)

    in_specs = [
        pl.BlockSpec((NC, BLK, H), lambda i: (0, i, 0)),
        pl.BlockSpec((BLK, H), lambda i: (i, 0)),
        pl.BlockSpec((BLK, 1), lambda i: (i, 0)),
    ]
    args = [S2, hs, dis]
    if has_prev:
        in_specs.append(pl.BlockSpec((BLK, H), lambda i: (i, 0)))
        args.append(prev)
    in_specs += [pl.BlockSpec((1, H), lambda i: (0, 0))] * 5
    args += [b, g, be, rm, rv]
    in_specs.append(pl.BlockSpec((H, H), lambda i: (0, 0)))
    args.append(Wn)

    return pl.pallas_call(
        body,
        grid=(NBLK,),
        in_specs=in_specs,
        out_specs=[
            pl.BlockSpec((BLK, H), lambda i: (i, 0)),
            pl.BlockSpec((BLK, H), lambda i: (i, 0)),
        ],
        out_shape=[
            jax.ShapeDtypeStruct((N, H), jnp.float32),
            jax.ShapeDtypeStruct((N, H), jnp.float32),
        ],
    )(*args)


def _tc_final(S2, hs3, dis, h2, b3, g3, be3, rm3, rv3, batch2, Wo, bo):
    def body(s2_ref, hs_ref, dis_ref, prev_ref, b_ref, g_ref, be_ref,
             rm_ref, rv_ref, bat_ref, wo_ref, bo_ref, out_ref,
             s_acc, cnt_acc, mx_acc):
        i = pl.program_id(0)

        @pl.when(i == 0)
        def _():
            s_acc[...] = jnp.zeros((G, H), jnp.float32)
            cnt_acc[...] = jnp.zeros((G, H), jnp.float32)
            mx_acc[...] = jnp.full((G, H), -jnp.inf, jnp.float32)

        dis = dis_ref[...]
        z = dis * (s2_ref[0] + s2_ref[1] + hs_ref[...]) + b_ref[...]
        a = g_ref[...] * lax.rsqrt(rv_ref[...] + 1e-5)
        cst = be_ref[...] - rm_ref[...] * a
        h = jnp.maximum(z * a + cst + prev_ref[...], 0.0)

        bat = bat_ref[...]
        gids = lax.broadcasted_iota(jnp.int32, (BLK, G), 1)
        oh = (bat == gids).astype(jnp.float32)
        dn = (((0,), (0,)), ((), ()))
        s_acc[...] = s_acc[...] + lax.dot_general(
            oh, h, dn, preferred_element_type=jnp.float32)
        cnt_acc[...] = cnt_acc[...] + lax.dot_general(
            oh, jnp.ones((BLK, H), jnp.float32), dn,
            preferred_element_type=jnp.float32)

        g_lo = jnp.min(bat)
        g_hi = jnp.max(bat)

        def mbody(gg, carry):
            m = jnp.max(jnp.where(bat == gg, h, -jnp.inf), axis=0,
                        keepdims=True)
            mx_acc[pl.ds(gg, 1), :] = jnp.maximum(mx_acc[pl.ds(gg, 1), :], m)
            return carry

        lax.fori_loop(g_lo, g_hi + 1, mbody, 0)

        @pl.when(i == NBLK - 1)
        def _():
            cnt = jnp.maximum(cnt_acc[...], 1.0)
            mean = s_acc[...] / cnt
            pooled = jnp.concatenate([mean, s_acc[...], mx_acc[...]], axis=1)
            out_ref[...] = jnp.dot(pooled, wo_ref[...],
                                   preferred_element_type=jnp.float32
                                   ) + bo_ref[...]

    return pl.pallas_call(
        body,
        grid=(NBLK,),
        in_specs=[
            pl.BlockSpec((NC, BLK, H), lambda i: (0, i, 0)),
            pl.BlockSpec((BLK, H), lambda i: (i, 0)),
            pl.BlockSpec((BLK, 1), lambda i: (i, 0)),
            pl.BlockSpec((BLK, H), lambda i: (i, 0)),
        ] + [pl.BlockSpec((1, H), lambda i: (0, 0))] * 5 + [
            pl.BlockSpec((BLK, 1), lambda i: (i, 0)),
            pl.BlockSpec((3 * H, C), lambda i: (0, 0)),
            pl.BlockSpec((1, C), lambda i: (0, 0)),
        ],
        out_specs=pl.BlockSpec((G, C), lambda i: (0, 0)),
        out_shape=jax.ShapeDtypeStruct((G, C), jnp.float32),
        scratch_shapes=[
            pltpu.VMEM((G, H), jnp.float32),
            pltpu.VMEM((G, H), jnp.float32),
            pltpu.VMEM((G, H), jnp.float32),
        ],
    )(S2, hs3, dis, h2, b3, g3, be3, rm3, rv3, batch2, Wo, bo)


def kernel(x, edge_index, edge_attr, batch, W1, b1, W2, b2, W3, b3,
           g1, be1, rm1, rv1, g2, be2, rm2, rv2, g3, be3, rm3, rv3, Wo, bo):
    del edge_attr
    zeros1 = jnp.zeros((RPT + TAIL,), jnp.float32)
    ones1 = jnp.ones((K + 8,), jnp.float32)
    zrows = jnp.zeros((ZR, D), jnp.float32)
    r2 = lambda v: v.reshape(1, -1)
    batch2 = batch.reshape(N, 1)
    src = edge_index[0]
    dst = edge_index[1]
    srcp = src
    dstp = dst

    deg2 = _sc_degree(dst, zeros1, ones1).reshape(NC, N, 1)
    dis, hs1 = _tc_pre(deg2, x, W1)
    S1 = _sc_scatter(hs1, srcp, dstp, zrows)
    h1, hs2 = _tc_mid(S1, hs1, dis, None, r2(b1), r2(g1), r2(be1),
                      r2(rm1), r2(rv1), W2, has_prev=False)
    S2 = _sc_scatter(hs2, srcp, dstp, zrows)
    h2, hs3 = _tc_mid(S2, hs2, dis, h1, r2(b2), r2(g2), r2(be2),
                      r2(rm2), r2(rv2), W3, has_prev=True)
    S3 = _sc_scatter(hs3, srcp, dstp, zrows)
    out = _tc_final(S3, hs3, dis, h2, r2(b3), r2(g3), r2(be3),
                    r2(rm3), r2(rv3), batch2, Wo, r2(bo))
    return out
